# baseline (speedup 1.0000x reference)
import functools

import jax
import jax.numpy as jnp
from jax import lax
from jax.experimental import pallas as pl
from jax.experimental.pallas import tpu as pltpu
from jax.experimental.pallas import tpu_sc as plsc

HIDDEN = 64
BLK = 128
NBUF = 6

_info = plsc.get_sparse_core_info()
NC, NS = _info.num_cores, _info.num_subcores
NW = NC * NS


def _transpose_block(src, dst, cvecs):
    def h_body(h, carry):
        hvec = jnp.full((16,), h, jnp.int32)
        for j in range(BLK // 16):
            v = plsc.load_gather(src, [cvecs[j], hvec])
            dst[h, pl.ds(j * 16, 16)] = v
        return carry

    lax.fori_loop(0, HIDDEN, h_body, 0)


def _run_job(tab_hbm, out_slices, nblk, idx_all, rows_v, vt_v,
             sems_g, sems_s, cvecs):
    def fire_gather(b, k):
        pltpu.async_copy(tab_hbm.at[idx_all.at[k]], rows_v.at[b], sems_g[b])

    def wait_gather(b):
        pltpu.make_async_copy(tab_hbm.at[idx_all.at[0]], rows_v.at[b],
                              sems_g[b]).wait()

    def fire_store(b, k):
        pltpu.async_copy(vt_v.at[b], out_slices(k), sems_s[b])

    def wait_store(b):
        pltpu.make_async_copy(vt_v.at[b], out_slices(0), sems_s[b]).wait()

    nb = min(NBUF, nblk)
    for b in range(nb):
        fire_gather(b, b)

    def group(g, carry):
        for b in range(NBUF):
            k = g * NBUF + b

            @pl.when(k < nblk)
            def _():
                wait_gather(b)

                @pl.when(k >= NBUF)
                def _():
                    wait_store(b)

                _transpose_block(rows_v.at[b], vt_v.at[b], cvecs)
                fire_store(b, k)

                @pl.when(k + NBUF < nblk)
                def _():
                    fire_gather(b, k + NBUF)

        return carry

    lax.fori_loop(0, -(-nblk // NBUF), group, 0)

    for b in range(nb):
        wait_store(b)


def _sc_kernel(n_gblk,
               user_i, traj_i, geo_i, ltraj_i, tgx_i, ggx_i,
               user_t, loc_t, geo_t,
               o_user, o_traj, o_geo, o_ltraj, o_tgx, o_ggx,
               idx_all, rows_v, vt_v, *sems):
    wid = lax.axis_index("s") * NC + lax.axis_index("c")
    sems_g, sems_s = sems[:NBUF], sems[NBUF:]
    cvecs = [lax.iota(jnp.int32, 16) + j * 16 for j in range(BLK // 16)]
    col0 = wid * BLK
    T = traj_i.shape[0]

    pltpu.sync_copy(user_i.at[pl.ds(col0, BLK)], idx_all.at[0, pl.ds(0, BLK)])
    _run_job(user_t, lambda k: o_user.at[:, pl.ds(col0, BLK)], 1,
             idx_all, rows_v, vt_v, sems_g, sems_s, cvecs)

    for idx_hbm, tab, out in ((traj_i, loc_t, o_traj),
                              (geo_i, geo_t, o_geo),
                              (ltraj_i, loc_t, o_ltraj)):
        pltpu.sync_copy(idx_hbm.at[:, pl.ds(col0, BLK)],
                        idx_all.at[pl.ds(0, T)])
        _run_job(tab, lambda k, out=out: out.at[k, :, pl.ds(col0, BLK)], T,
                 idx_all, rows_v, vt_v, sems_g, sems_s, cvecs)

    GB = 13
    start = jnp.minimum(wid * GB, n_gblk - GB)
    for idx_hbm, tab, out in ((tgx_i, loc_t, o_tgx), (ggx_i, geo_t, o_ggx)):
        pltpu.sync_copy(idx_hbm.at[pl.ds(start, GB)], idx_all.at[pl.ds(0, GB)])
        _run_job(tab,
                 lambda k, out=out: out.at[:, pl.ds((start + k) * BLK, BLK)],
                 GB, idx_all, rows_v, vt_v, sems_g, sems_s, cvecs)


def kernel(user, traj, geo, long_traj, traj_graph_x, geo_graph_x,
           user_table, loc_table, geo_table):
    B, T = traj.shape
    n_graph = traj_graph_x.shape[0]
    n_graph_pad = -(-n_graph // BLK) * BLK
    n_gblk = n_graph_pad // BLK

    tT = traj.T
    gT = geo.T
    ltT = long_traj.T
    tgx2 = jnp.pad(traj_graph_x, (0, n_graph_pad - n_graph)).reshape(-1, BLK)
    ggx2 = jnp.pad(geo_graph_x, (0, n_graph_pad - n_graph)).reshape(-1, BLK)

    mesh = plsc.VectorSubcoreMesh(core_axis_name="c", subcore_axis_name="s")
    f = pl.kernel(
        functools.partial(_sc_kernel, n_gblk),
        mesh=mesh,
        compiler_params=pltpu.CompilerParams(use_tc_tiling_on_sc=False,
                                             needs_layout_passes=False),
        out_type=[
            jax.ShapeDtypeStruct((HIDDEN, B), jnp.float32),
            jax.ShapeDtypeStruct((T, HIDDEN, B), jnp.float32),
            jax.ShapeDtypeStruct((T, HIDDEN, B), jnp.float32),
            jax.ShapeDtypeStruct((T, HIDDEN, B), jnp.float32),
            jax.ShapeDtypeStruct((HIDDEN, n_graph_pad), jnp.float32),
            jax.ShapeDtypeStruct((HIDDEN, n_graph_pad), jnp.float32),
        ],
        scratch_types=[
            pltpu.VMEM((T, BLK), jnp.int32),
            pltpu.VMEM((NBUF, BLK, HIDDEN), jnp.float32),
            pltpu.VMEM((NBUF, HIDDEN, BLK), jnp.float32),
        ] + [pltpu.SemaphoreType.DMA] * (2 * NBUF),
    )
    o_user, o_traj, o_geo, o_ltraj, o_tgx, o_ggx = f(
        user, tT, gT, ltT, tgx2, ggx2, user_table, loc_table, geo_table)
    return (
        o_user.T,
        o_traj.transpose(2, 0, 1),
        o_geo.transpose(2, 0, 1),
        o_ltraj.transpose(2, 0, 1),
        o_tgx.T[:n_graph],
        o_ggx.T[:n_graph],
    )


# speedup vs baseline: 4.5271x; 4.5271x over previous
import functools

import jax
import jax.numpy as jnp
from jax import lax
from jax.experimental import pallas as pl
from jax.experimental.pallas import tpu as pltpu
from jax.experimental.pallas import tpu_sc as plsc

HIDDEN = 64
BLK = 128
NBUF = 6

_info = plsc.get_sparse_core_info()
NC, NS = _info.num_cores, _info.num_subcores
NW = NC * NS


def _transpose_block(src, dst, idxv):
    hgv, hrv, zv = idxv

    @plsc.parallel_loop(0, BLK, unroll=4)
    def c_body(c):
        cvec = jnp.full((16,), c, jnp.int32)
        for g in range(HIDDEN // 16):
            v = src[c, pl.ds(g * 16, 16)]
            plsc.store_scatter(dst, [hgv + 2 * g, zv, hrv, cvec], v)


def _run_job(tab_hbm, out_slices, nblk, idx_all, rows_v, vt_v,
             sems_g, sems_s, idxv):
    def fire_gather(b, k):
        pltpu.async_copy(tab_hbm.at[idx_all.at[k]], rows_v.at[b], sems_g[b])

    def wait_gather(b):
        pltpu.make_async_copy(tab_hbm.at[idx_all.at[0]], rows_v.at[b],
                              sems_g[b]).wait()

    def fire_store(b, k):
        pltpu.async_copy(vt_v.at[b, :, :, :, pl.ds(0, BLK)], out_slices(k),
                         sems_s[b])

    def wait_store(b):
        pltpu.make_async_copy(vt_v.at[b, :, :, :, pl.ds(0, BLK)],
                              out_slices(0), sems_s[b]).wait()

    nb = min(NBUF, nblk)
    for b in range(nb):
        fire_gather(b, b)

    def group(g, carry):
        for b in range(NBUF):
            k = g * NBUF + b

            @pl.when(k < nblk)
            def _():
                wait_gather(b)

                @pl.when(k >= NBUF)
                def _():
                    wait_store(b)

                _transpose_block(rows_v.at[b], vt_v.at[b], idxv)
                fire_store(b, k)

                @pl.when(k + NBUF < nblk)
                def _():
                    fire_gather(b, k + NBUF)

        return carry

    lax.fori_loop(0, -(-nblk // NBUF), group, 0)

    for b in range(nb):
        wait_store(b)


def _sc_kernel(n_gblk,
               user_i, traj_i, geo_i, ltraj_i, tgx_i, ggx_i,
               user_t, loc_t, geo_t,
               o_user, o_traj, o_geo, o_ltraj, o_tgx, o_ggx,
               idx_all, rows_v, vt_v, *sems):
    wid = lax.axis_index("s") * NC + lax.axis_index("c")
    sems_g, sems_s = sems[:NBUF], sems[NBUF:]
    lane = lax.iota(jnp.int32, 16)
    idxv = (lane // 8, lane % 8, lane * 0)
    col0 = wid * BLK
    T = traj_i.shape[0]

    pltpu.sync_copy(user_i.at[pl.ds(col0, BLK)], idx_all.at[0, pl.ds(0, BLK)])
    _run_job(user_t, lambda k: o_user.at[:, pl.ds(wid, 1)], 1,
             idx_all, rows_v, vt_v, sems_g, sems_s, idxv)

    for idx_hbm, tab, out in ((traj_i, loc_t, o_traj),
                              (geo_i, geo_t, o_geo),
                              (ltraj_i, loc_t, o_ltraj)):
        pltpu.sync_copy(idx_hbm.at[:, pl.ds(col0, BLK)],
                        idx_all.at[pl.ds(0, T)])
        _run_job(tab, lambda k, out=out: out.at[k, :, pl.ds(wid, 1)], T,
                 idx_all, rows_v, vt_v, sems_g, sems_s, idxv)

    GB = 13
    start = jnp.minimum(wid * GB, n_gblk - GB)
    for idx_hbm, tab, out in ((tgx_i, loc_t, o_tgx), (ggx_i, geo_t, o_ggx)):
        pltpu.sync_copy(idx_hbm.at[pl.ds(start, GB)], idx_all.at[pl.ds(0, GB)])
        _run_job(tab,
                 lambda k, out=out: out.at[:, pl.ds(start + k, 1)],
                 GB, idx_all, rows_v, vt_v, sems_g, sems_s, idxv)


def kernel(user, traj, geo, long_traj, traj_graph_x, geo_graph_x,
           user_table, loc_table, geo_table):
    B, T = traj.shape
    n_graph = traj_graph_x.shape[0]
    n_graph_pad = -(-n_graph // BLK) * BLK
    n_gblk = n_graph_pad // BLK
    NB = B // BLK
    HG = HIDDEN // 8

    tT = traj.T
    gT = geo.T
    ltT = long_traj.T
    tgx2 = jnp.pad(traj_graph_x, (0, n_graph_pad - n_graph)).reshape(-1, BLK)
    ggx2 = jnp.pad(geo_graph_x, (0, n_graph_pad - n_graph)).reshape(-1, BLK)

    mesh = plsc.VectorSubcoreMesh(core_axis_name="c", subcore_axis_name="s")
    f = pl.kernel(
        functools.partial(_sc_kernel, n_gblk),
        mesh=mesh,
        compiler_params=pltpu.CompilerParams(use_tc_tiling_on_sc=False,
                                             needs_layout_passes=False),
        out_type=[
            jax.ShapeDtypeStruct((HG, NB, 8, BLK), jnp.float32),
            jax.ShapeDtypeStruct((T, HG, NB, 8, BLK), jnp.float32),
            jax.ShapeDtypeStruct((T, HG, NB, 8, BLK), jnp.float32),
            jax.ShapeDtypeStruct((T, HG, NB, 8, BLK), jnp.float32),
            jax.ShapeDtypeStruct((HG, n_gblk, 8, BLK), jnp.float32),
            jax.ShapeDtypeStruct((HG, n_gblk, 8, BLK), jnp.float32),
        ],
        scratch_types=[
            pltpu.VMEM((T, BLK), jnp.int32),
            pltpu.VMEM((NBUF, BLK, HIDDEN), jnp.float32),
            pltpu.VMEM((NBUF, HG, 1, 8, BLK + 1), jnp.float32),
        ] + [pltpu.SemaphoreType.DMA] * (2 * NBUF),
    )
    o_user, o_traj, o_geo, o_ltraj, o_tgx, o_ggx = f(
        user, tT, gT, ltT, tgx2, ggx2, user_table, loc_table, geo_table)

    def unt2(o):
        return o.transpose(1, 3, 0, 2).reshape(-1, HIDDEN)

    def unt3(o):
        return o.transpose(2, 4, 0, 1, 3).reshape(B, T, HIDDEN)

    return (
        unt2(o_user),
        unt3(o_traj),
        unt3(o_geo),
        unt3(o_ltraj),
        unt2(o_tgx)[:n_graph],
        unt2(o_ggx)[:n_graph],
    )


# speedup vs baseline: 4.7235x; 1.0434x over previous
import functools

import jax
import jax.numpy as jnp
from jax import lax
from jax.experimental import pallas as pl
from jax.experimental.pallas import tpu as pltpu
from jax.experimental.pallas import tpu_sc as plsc

HIDDEN = 64
BLK = 128
NBUF = 6

_info = plsc.get_sparse_core_info()
NC, NS = _info.num_cores, _info.num_subcores
NW = NC * NS


def _transpose_block(src, dst, idxv):
    hgv, hrv, zv = idxv

    @plsc.parallel_loop(0, BLK, unroll=4)
    def c_body(c):
        cvec = jnp.full((16,), c, jnp.int32)
        for g in range(HIDDEN // 16):
            v = src[c, pl.ds(g * 16, 16)]
            plsc.store_scatter(dst, [hgv + 2 * g, zv, hrv, cvec], v)


def _run_job(tab_hbm, out_slices, nblk, idx_all, rows_v, vt_v,
             sems_g, sems_s, idxv):
    def fire_gather(b, k):
        pltpu.async_copy(tab_hbm.at[idx_all.at[k]], rows_v.at[b], sems_g[b])

    def wait_gather(b):
        pltpu.make_async_copy(tab_hbm.at[idx_all.at[0]], rows_v.at[b],
                              sems_g[b]).wait()

    def fire_store(b, k):
        pltpu.async_copy(vt_v.at[b, :, :, :, pl.ds(0, BLK)], out_slices(k),
                         sems_s[b])

    def wait_store(b):
        pltpu.make_async_copy(vt_v.at[b, :, :, :, pl.ds(0, BLK)],
                              out_slices(0), sems_s[b]).wait()

    nb = min(NBUF, nblk)
    for b in range(nb):
        fire_gather(b, b)

    def group(g, carry):
        for b in range(NBUF):
            k = g * NBUF + b

            @pl.when(k < nblk)
            def _():
                wait_gather(b)

                @pl.when(k >= NBUF)
                def _():
                    wait_store(b)

                _transpose_block(rows_v.at[b], vt_v.at[b], idxv)
                fire_store(b, k)

                @pl.when(k + NBUF < nblk)
                def _():
                    fire_gather(b, k + NBUF)

        return carry

    lax.fori_loop(0, -(-nblk // NBUF), group, 0)

    for b in range(nb):
        wait_store(b)


def _sc_kernel_a(n_gblk,
                 user_i, geo_i, ggx_i, user_t, geo_t,
                 o_user, o_geo, o_ggx,
                 idx_all, rows_v, vt_v, *sems):
    wid = lax.axis_index("s") * NC + lax.axis_index("c")
    sems_g, sems_s = sems[:NBUF], sems[NBUF:]
    lane = lax.iota(jnp.int32, 16)
    idxv = (lane // 8, lane % 8, lane * 0)
    col0 = wid * BLK
    T = geo_i.shape[0]

    pltpu.sync_copy(user_i.at[pl.ds(col0, BLK)], idx_all.at[0, pl.ds(0, BLK)])
    _run_job(user_t, lambda k: o_user.at[:, pl.ds(wid, 1)], 1,
             idx_all, rows_v, vt_v, sems_g, sems_s, idxv)

    pltpu.sync_copy(geo_i.at[:, pl.ds(col0, BLK)], idx_all.at[pl.ds(0, T)])
    _run_job(geo_t, lambda k: o_geo.at[k, :, pl.ds(wid, 1)], T,
             idx_all, rows_v, vt_v, sems_g, sems_s, idxv)

    GB = 13
    start = jnp.minimum(wid * GB, n_gblk - GB)
    pltpu.sync_copy(ggx_i.at[pl.ds(start, GB)], idx_all.at[pl.ds(0, GB)])
    _run_job(geo_t, lambda k: o_ggx.at[:, pl.ds(start + k, 1)],
             GB, idx_all, rows_v, vt_v, sems_g, sems_s, idxv)


def _sc_kernel_b(n_gblk,
                 traj_i, ltraj_i, tgx_i, loc_t,
                 o_traj, o_ltraj, o_tgx,
                 idx_all, rows_v, vt_v, *sems):
    wid = lax.axis_index("s") * NC + lax.axis_index("c")
    sems_g, sems_s = sems[:NBUF], sems[NBUF:]
    lane = lax.iota(jnp.int32, 16)
    idxv = (lane // 8, lane % 8, lane * 0)
    col0 = wid * BLK
    T = traj_i.shape[0]

    for idx_hbm, out in ((traj_i, o_traj), (ltraj_i, o_ltraj)):
        pltpu.sync_copy(idx_hbm.at[:, pl.ds(col0, BLK)],
                        idx_all.at[pl.ds(0, T)])
        _run_job(loc_t, lambda k, out=out: out.at[k, :, pl.ds(wid, 1)], T,
                 idx_all, rows_v, vt_v, sems_g, sems_s, idxv)

    GB = 13
    start = jnp.minimum(wid * GB, n_gblk - GB)
    pltpu.sync_copy(tgx_i.at[pl.ds(start, GB)], idx_all.at[pl.ds(0, GB)])
    _run_job(loc_t, lambda k: o_tgx.at[:, pl.ds(start + k, 1)],
             GB, idx_all, rows_v, vt_v, sems_g, sems_s, idxv)


def kernel(user, traj, geo, long_traj, traj_graph_x, geo_graph_x,
           user_table, loc_table, geo_table):
    B, T = traj.shape
    n_graph = traj_graph_x.shape[0]
    n_graph_pad = -(-n_graph // BLK) * BLK
    n_gblk = n_graph_pad // BLK
    NB = B // BLK
    HG = HIDDEN // 8

    tT = traj.T
    gT = geo.T
    ltT = long_traj.T
    tgx2 = jnp.pad(traj_graph_x, (0, n_graph_pad - n_graph)).reshape(-1, BLK)
    ggx2 = jnp.pad(geo_graph_x, (0, n_graph_pad - n_graph)).reshape(-1, BLK)

    mesh = plsc.VectorSubcoreMesh(core_axis_name="c", subcore_axis_name="s")
    cparams = pltpu.CompilerParams(use_tc_tiling_on_sc=False,
                                   needs_layout_passes=False)
    scratch = [
        pltpu.VMEM((T, BLK), jnp.int32),
        pltpu.VMEM((NBUF, BLK, HIDDEN), jnp.float32),
        pltpu.VMEM((NBUF, HG, 1, 8, BLK + 1), jnp.float32),
    ] + [pltpu.SemaphoreType.DMA] * (2 * NBUF)

    fa = pl.kernel(
        functools.partial(_sc_kernel_a, n_gblk),
        mesh=mesh,
        compiler_params=cparams,
        out_type=[
            jax.ShapeDtypeStruct((HG, NB, 8, BLK), jnp.float32),
            jax.ShapeDtypeStruct((T, HG, NB, 8, BLK), jnp.float32),
            jax.ShapeDtypeStruct((HG, n_gblk, 8, BLK), jnp.float32),
        ],
        scratch_types=scratch,
    )
    fb = pl.kernel(
        functools.partial(_sc_kernel_b, n_gblk),
        mesh=mesh,
        compiler_params=cparams,
        out_type=[
            jax.ShapeDtypeStruct((T, HG, NB, 8, BLK), jnp.float32),
            jax.ShapeDtypeStruct((T, HG, NB, 8, BLK), jnp.float32),
            jax.ShapeDtypeStruct((HG, n_gblk, 8, BLK), jnp.float32),
        ],
        scratch_types=scratch,
    )
    o_user, o_geo, o_ggx = fa(user, gT, ggx2, user_table, geo_table)
    o_traj, o_ltraj, o_tgx = fb(tT, ltT, tgx2, loc_table)

    def unt2(o):
        return o.transpose(1, 3, 0, 2).reshape(-1, HIDDEN)

    def unt3(o):
        return o.transpose(2, 4, 0, 1, 3).reshape(B, T, HIDDEN)

    return (
        unt2(o_user),
        unt3(o_traj),
        unt3(o_geo),
        unt3(o_ltraj),
        unt2(o_tgx)[:n_graph],
        unt2(o_ggx)[:n_graph],
    )


# speedup vs baseline: 4.7759x; 1.0111x over previous
import functools

import jax
import jax.numpy as jnp
from jax import lax
from jax.experimental import pallas as pl
from jax.experimental.pallas import tpu as pltpu
from jax.experimental.pallas import tpu_sc as plsc

HIDDEN = 64
BLK = 128
NBUF = 6

_info = plsc.get_sparse_core_info()
NC, NS = _info.num_cores, _info.num_subcores
NW = NC * NS


def _transpose_block(src, dst, idxv):
    hgv, hrv, zv = idxv

    @plsc.parallel_loop(0, BLK, unroll=4)
    def c_body(c):
        cvec = jnp.full((16,), c, jnp.int32)
        for g in range(HIDDEN // 16):
            v = src[c, pl.ds(g * 16, 16)]
            plsc.store_scatter(dst, [hgv + 2 * g, zv, hrv, cvec], v)


def _run_job(tab_hbm, out_slices, nblk, idx_all, rows_v, vt_v,
             sems_g, sems_s, idxv):
    def fire_gather(b, k):
        pltpu.async_copy(tab_hbm.at[idx_all.at[k]], rows_v.at[b], sems_g[b])

    def wait_gather(b):
        pltpu.make_async_copy(tab_hbm.at[idx_all.at[0]], rows_v.at[b],
                              sems_g[b]).wait()

    def fire_store(b, k):
        pltpu.async_copy(vt_v.at[b, :, :, :, pl.ds(0, BLK)], out_slices(k),
                         sems_s[b])

    def wait_store(b):
        pltpu.make_async_copy(vt_v.at[b, :, :, :, pl.ds(0, BLK)],
                              out_slices(0), sems_s[b]).wait()

    nb = min(NBUF, nblk)
    for b in range(nb):
        fire_gather(b, b)

    def group(g, carry):
        for b in range(NBUF):
            k = g * NBUF + b

            @pl.when(k < nblk)
            def _():
                wait_gather(b)

                @pl.when(k >= NBUF)
                def _():
                    wait_store(b)

                _transpose_block(rows_v.at[b], vt_v.at[b], idxv)
                fire_store(b, k)

                @pl.when(k + NBUF < nblk)
                def _():
                    fire_gather(b, k + NBUF)

        return carry

    lax.fori_loop(0, -(-nblk // NBUF), group, 0)

    for b in range(nb):
        wait_store(b)


def _sc_kernel_a(n_gblk,
                 user_i, geo_i, ggx_i, user_t, geo_t,
                 o_user, o_geo, o_ggx,
                 idx_all, rows_v, vt_v, *sems):
    wid = lax.axis_index("s") * NC + lax.axis_index("c")
    sems_g, sems_s = sems[:NBUF], sems[NBUF:]
    lane = lax.iota(jnp.int32, 16)
    idxv = (lane // 8, lane % 8, lane * 0)
    col0 = wid * BLK
    T = geo_i.shape[0]

    pltpu.sync_copy(user_i.at[pl.ds(col0, BLK)], idx_all.at[0, pl.ds(0, BLK)])
    _run_job(user_t, lambda k: o_user.at[:, pl.ds(wid, 1)], 1,
             idx_all, rows_v, vt_v, sems_g, sems_s, idxv)

    pltpu.sync_copy(geo_i.at[:, pl.ds(col0, BLK)], idx_all.at[pl.ds(0, T)])
    _run_job(geo_t, lambda k: o_geo.at[k, :, pl.ds(wid, 1)], T,
             idx_all, rows_v, vt_v, sems_g, sems_s, idxv)

    GB = 13
    start = jnp.minimum(wid * GB, n_gblk - GB)
    pltpu.sync_copy(ggx_i.at[pl.ds(start, GB)], idx_all.at[pl.ds(0, GB)])
    _run_job(geo_t, lambda k: o_ggx.at[:, pl.ds(start + k, 1)],
             GB, idx_all, rows_v, vt_v, sems_g, sems_s, idxv)


def _sc_kernel_b(n_gblk,
                 traj_i, ltraj_i, tgx_i, loc_t,
                 o_traj, o_ltraj, o_tgx,
                 idx_all, rows_v, vt_v, *sems):
    wid = lax.axis_index("s") * NC + lax.axis_index("c")
    sems_g, sems_s = sems[:NBUF], sems[NBUF:]
    lane = lax.iota(jnp.int32, 16)
    idxv = (lane // 8, lane % 8, lane * 0)
    col0 = wid * BLK
    T = traj_i.shape[0]

    for idx_hbm, out in ((traj_i, o_traj), (ltraj_i, o_ltraj)):
        pltpu.sync_copy(idx_hbm.at[:, pl.ds(col0, BLK)],
                        idx_all.at[pl.ds(0, T)])
        _run_job(loc_t, lambda k, out=out: out.at[k, :, pl.ds(wid, 1)], T,
                 idx_all, rows_v, vt_v, sems_g, sems_s, idxv)

    GB = 13
    start = jnp.minimum(wid * GB, n_gblk - GB)
    pltpu.sync_copy(tgx_i.at[pl.ds(start, GB)], idx_all.at[pl.ds(0, GB)])
    _run_job(loc_t, lambda k: o_tgx.at[:, pl.ds(start + k, 1)],
             GB, idx_all, rows_v, vt_v, sems_g, sems_s, idxv)


def kernel(user, traj, geo, long_traj, traj_graph_x, geo_graph_x,
           user_table, loc_table, geo_table):
    B, T = traj.shape
    n_graph = traj_graph_x.shape[0]
    n_graph_pad = -(-n_graph // BLK) * BLK
    n_gblk = n_graph_pad // BLK
    NB = B // BLK
    HG = HIDDEN // 8

    user_p = jnp.pad(user_table, ((0, 0), (0, HIDDEN))).reshape(-1, HIDDEN)
    loc_p = jnp.pad(loc_table, ((0, 0), (0, HIDDEN))).reshape(-1, HIDDEN)
    geo_p = jnp.pad(geo_table, ((0, 0), (0, HIDDEN))).reshape(-1, HIDDEN)

    tT = traj.T * 2
    gT = geo.T * 2
    ltT = long_traj.T * 2
    user2 = user * 2
    tgx2 = (jnp.pad(traj_graph_x, (0, n_graph_pad - n_graph)) * 2
            ).reshape(-1, BLK)
    ggx2 = (jnp.pad(geo_graph_x, (0, n_graph_pad - n_graph)) * 2
            ).reshape(-1, BLK)

    mesh = plsc.VectorSubcoreMesh(core_axis_name="c", subcore_axis_name="s")
    cparams = pltpu.CompilerParams(use_tc_tiling_on_sc=False,
                                   needs_layout_passes=False)
    scratch = [
        pltpu.VMEM((T, BLK), jnp.int32),
        pltpu.VMEM((NBUF, BLK, HIDDEN), jnp.float32),
        pltpu.VMEM((NBUF, HG, 1, 8, BLK + 1), jnp.float32),
    ] + [pltpu.SemaphoreType.DMA] * (2 * NBUF)

    fa = pl.kernel(
        functools.partial(_sc_kernel_a, n_gblk),
        mesh=mesh,
        compiler_params=cparams,
        out_type=[
            jax.ShapeDtypeStruct((HG, NB, 8, BLK), jnp.float32),
            jax.ShapeDtypeStruct((T, HG, NB, 8, BLK), jnp.float32),
            jax.ShapeDtypeStruct((HG, n_gblk, 8, BLK), jnp.float32),
        ],
        scratch_types=scratch,
    )
    fb = pl.kernel(
        functools.partial(_sc_kernel_b, n_gblk),
        mesh=mesh,
        compiler_params=cparams,
        out_type=[
            jax.ShapeDtypeStruct((T, HG, NB, 8, BLK), jnp.float32),
            jax.ShapeDtypeStruct((T, HG, NB, 8, BLK), jnp.float32),
            jax.ShapeDtypeStruct((HG, n_gblk, 8, BLK), jnp.float32),
        ],
        scratch_types=scratch,
    )
    o_user, o_geo, o_ggx = fa(user2, gT, ggx2, user_p, geo_p)
    o_traj, o_ltraj, o_tgx = fb(tT, ltT, tgx2, loc_p)

    def unt2(o):
        return o.transpose(1, 3, 0, 2).reshape(-1, HIDDEN)

    def unt3(o):
        return o.transpose(2, 4, 0, 1, 3).reshape(B, T, HIDDEN)

    return (
        unt2(o_user),
        unt3(o_traj),
        unt3(o_geo),
        unt3(o_ltraj),
        unt2(o_tgx)[:n_graph],
        unt2(o_ggx)[:n_graph],
    )
